# baseline (speedup 1.0000x reference)
import functools

import jax
import jax.numpy as jnp
from jax import lax
from jax.experimental import pallas as pl
from jax.experimental.pallas import tpu as pltpu
from jax.experimental.pallas import tpu_sc as plsc

_NC = 2
_NS = 16
_NW = _NC * _NS
_CHUNK = 512


@jax.jit
def _gather_rows(table, idx):
    n = idx.shape[0]
    d = table.shape[1]
    b_per_w = n // _NW
    n_chunks = b_per_w // _CHUNK
    mesh = plsc.VectorSubcoreMesh(
        core_axis_name="c", subcore_axis_name="s", num_cores=_NC, num_subcores=_NS
    )

    @functools.partial(
        pl.kernel,
        out_type=jax.ShapeDtypeStruct((n, d), jnp.float32),
        mesh=mesh,
        scratch_types=[
            pltpu.VMEM((b_per_w,), jnp.int32),
            pltpu.VMEM((_CHUNK, d), jnp.float32),
            pltpu.SemaphoreType.DMA,
        ],
        compiler_params=pltpu.CompilerParams(use_tc_tiling_on_sc=False),
    )
    def k(table_hbm, idx_hbm, out_hbm, idx_v, rows_v, gsem):
        wid = lax.axis_index("s") * _NC + lax.axis_index("c")
        base = wid * b_per_w
        pltpu.sync_copy(idx_hbm.at[pl.ds(base, b_per_w)], idx_v)

        def body(i, carry):
            off = i * _CHUNK
            g = pltpu.async_copy(
                table_hbm.at[idx_v.at[pl.ds(off, _CHUNK)]], rows_v, gsem
            )
            g.wait()
            pltpu.sync_copy(rows_v, out_hbm.at[pl.ds(base + off, _CHUNK)])
            return carry

        lax.fori_loop(0, n_chunks, body, 0)

    return k(table, idx)


def kernel(x, table):
    b, s = x.shape
    d = table.shape[1]
    out = _gather_rows(table, x.reshape(b * s))
    return out.reshape(b, s, d)


# speedup vs baseline: 1.0210x; 1.0210x over previous
import functools

import jax
import jax.numpy as jnp
from jax import lax
from jax.experimental import pallas as pl
from jax.experimental.pallas import tpu as pltpu
from jax.experimental.pallas import tpu_sc as plsc

_NC = 2
_NS = 16
_NW = _NC * _NS
_CHUNK = 800


@jax.jit
def _gather_rows(table, idx):
    n = idx.shape[0]
    d = table.shape[1]
    b_per_w = n // _NW
    n_chunks = b_per_w // _CHUNK
    mesh = plsc.VectorSubcoreMesh(
        core_axis_name="c", subcore_axis_name="s", num_cores=_NC, num_subcores=_NS
    )

    @functools.partial(
        pl.kernel,
        out_type=jax.ShapeDtypeStruct((n, d), jnp.float32),
        mesh=mesh,
        scratch_types=[
            pltpu.VMEM((b_per_w,), jnp.int32),
            pltpu.VMEM((2, _CHUNK, d), jnp.float32),
            pltpu.SemaphoreType.DMA,
            pltpu.SemaphoreType.DMA,
        ],
        compiler_params=pltpu.CompilerParams(use_tc_tiling_on_sc=False),
    )
    def k(table_hbm, idx_hbm, out_hbm, idx_v, rows_v, gsem0, gsem1):
        wid = lax.axis_index("s") * _NC + lax.axis_index("c")
        base = wid * b_per_w
        pltpu.sync_copy(idx_hbm.at[pl.ds(base, b_per_w)], idx_v)

        def start_gather(i, buf, sem):
            off = i * _CHUNK
            pltpu.async_copy(
                table_hbm.at[idx_v.at[pl.ds(off, _CHUNK)]], rows_v.at[buf], sem
            )

        def wait_gather(i, buf, sem):
            off = i * _CHUNK
            pltpu.make_async_copy(
                table_hbm.at[idx_v.at[pl.ds(off, _CHUNK)]], rows_v.at[buf], sem
            ).wait()

        start_gather(0, 0, gsem0)

        def body(i, carry):
            @pl.when(i + 1 < n_chunks)
            def _():
                @pl.when(lax.rem(i + 1, 2) == 0)
                def _():
                    start_gather(i + 1, 0, gsem0)

                @pl.when(lax.rem(i + 1, 2) == 1)
                def _():
                    start_gather(i + 1, 1, gsem1)

            @pl.when(lax.rem(i, 2) == 0)
            def _():
                wait_gather(i, 0, gsem0)
                pltpu.sync_copy(rows_v.at[0], out_hbm.at[pl.ds(base + i * _CHUNK, _CHUNK)])

            @pl.when(lax.rem(i, 2) == 1)
            def _():
                wait_gather(i, 1, gsem1)
                pltpu.sync_copy(rows_v.at[1], out_hbm.at[pl.ds(base + i * _CHUNK, _CHUNK)])

            return carry

        lax.fori_loop(0, n_chunks, body, 0)

    return k(table, idx)


def kernel(x, table):
    b, s = x.shape
    d = table.shape[1]
    out = _gather_rows(table, x.reshape(b * s))
    return out.reshape(b, s, d)
